# baseline (speedup 1.0000x reference)
import functools

import jax
import jax.numpy as jnp
from jax import lax
from jax.experimental import pallas as pl
from jax.experimental.pallas import tpu as pltpu
from jax.experimental.pallas import tpu_sc as plsc

F32 = jnp.float32
I32 = jnp.int32

_NS = 16
_NC = 2
_B = 128


def _sc_mesh():
    return plsc.VectorSubcoreMesh(core_axis_name="c", subcore_axis_name="s")


def _make_deg_kernel(n_pad, e_pad):
    rows_tile = n_pad // _NS
    nb = e_pad // (_NC * _NS * _B)

    @functools.partial(
        pl.kernel,
        out_type=jax.ShapeDtypeStruct((_NC * n_pad,), F32),
        mesh=_sc_mesh(),
        scratch_types=[
            pltpu.VMEM((nb, _B), I32),
            pltpu.VMEM((_B,), F32),
            pltpu.VMEM((rows_tile,), F32),
            pltpu.VMEM_SHARED((n_pad,), F32),
        ],
    )
    def deg_kernel(dst_hbm, out_hbm, dstv, ones, zbuf, acc):
        cid = lax.axis_index("c")
        sid = lax.axis_index("s")
        wid = cid * _NS + sid

        def fill_ones(i, _):
            ones[pl.ds(i * 16, 16)] = jnp.ones((16,), F32)
            return _
        lax.fori_loop(0, _B // 16, fill_ones, None)

        def fill_z(i, _):
            zbuf[pl.ds(i * 16, 16)] = jnp.zeros((16,), F32)
            return _
        lax.fori_loop(0, rows_tile // 16, fill_z, None)
        pltpu.sync_copy(zbuf, acc.at[pl.ds(sid * rows_tile, rows_tile)])
        plsc.subcore_barrier()

        pltpu.sync_copy(dst_hbm.at[pl.ds(wid * nb, nb)], dstv)

        def scat(j, _):
            pltpu.sync_copy(ones, acc.at[dstv.at[j]], add=True)
            return _
        lax.fori_loop(0, nb, scat, None)
        plsc.subcore_barrier()

        off = cid * n_pad + sid * rows_tile
        pltpu.sync_copy(acc.at[pl.ds(sid * rows_tile, rows_tile)],
                        out_hbm.at[pl.ds(off, rows_tile)])

    return deg_kernel


def _make_agg_kernel(n, n_pad, e_pad):
    rows_acc = n_pad // _NS
    nb = e_pad // (_NS * _B)
    nh = nb // 2
    zrows = 128
    nz = rows_acc // zrows

    @functools.partial(
        pl.kernel,
        out_type=jax.ShapeDtypeStruct((4, n_pad, 64), F32),
        mesh=_sc_mesh(),
        compiler_params=pltpu.CompilerParams(use_tc_tiling_on_sc=False),
        scratch_types=[
            pltpu.VMEM((nh, _B), I32),
            pltpu.VMEM((nh, _B), I32),
            pltpu.VMEM((_B, 64), F32),
            pltpu.VMEM((_B, 64), F32),
            pltpu.VMEM((_B, 64), F32),
            pltpu.VMEM((_B, 64), F32),
            pltpu.VMEM_SHARED((n_pad, 64), F32),
            pltpu.VMEM_SHARED((n_pad, 64), F32),
            [pltpu.SemaphoreType.DMA] * 4,
            [pltpu.SemaphoreType.DMA] * 4,
        ],
    )
    def agg_kernel(hs_hbm, src_hbm, dst_hbm, out_hbm, srcv, dstv, rows0,
                   rows1, rows2, rows3, table, acc, sg, ss):
        cid = lax.axis_index("c")
        sid = lax.axis_index("s")

        for q in range(2):
            qq = cid * 2 + q

            pltpu.sync_copy(
                hs_hbm.at[qq, pl.ds(sid * rows_acc, rows_acc)],
                table.at[pl.ds(sid * rows_acc, rows_acc)])

            def fz(k, _):
                j = k // 4
                i = k - j * 4
                rows0[j, pl.ds(i * 16, 16)] = jnp.zeros((16,), F32)
                return _
            lax.fori_loop(0, zrows * 4, fz, None)

            def zc(t, _):
                pltpu.sync_copy(rows0,
                                acc.at[pl.ds(sid * rows_acc + t * zrows, zrows)])
                return _
            lax.fori_loop(0, nz, zc, None)
            plsc.subcore_barrier()

            bufs = (rows0, rows1, rows2, rows3)

            def half_loop(hf, _):
                base = sid * nb + hf * nh
                pltpu.sync_copy(src_hbm.at[pl.ds(base, nh)], srcv)
                pltpu.sync_copy(dst_hbm.at[pl.ds(base, nh)], dstv)

                pltpu.async_copy(table.at[srcv.at[0]], rows0, sg[0])
                pltpu.async_copy(table.at[srcv.at[1]], rows1, sg[1])

                def quad(g, __):
                    for p in range(4):
                        j = 4 * g + p
                        q = (p + 2) % 4
                        pltpu.make_async_copy(
                            table.at[srcv.at[j]], bufs[p], sg[p]).wait()
                        pltpu.async_copy(
                            bufs[p], acc.at[dstv.at[j]], ss[p], add=True)

                        @pl.when(j >= 2)
                        def _wait_prev():
                            pltpu.make_async_copy(
                                bufs[q], acc.at[dstv.at[j]], ss[q]).wait()

                        @pl.when(j + 2 < nh)
                        def _issue_next():
                            pltpu.async_copy(
                                table.at[srcv.at[j + 2]], bufs[q], sg[q])
                    return __
                lax.fori_loop(0, nh // 4, quad, None)
                pltpu.make_async_copy(
                    bufs[(nh - 2) % 4], acc.at[dstv.at[nh - 2]], ss[(nh - 2) % 4]).wait()
                pltpu.make_async_copy(
                    bufs[(nh - 1) % 4], acc.at[dstv.at[nh - 1]], ss[(nh - 1) % 4]).wait()
                return _
            lax.fori_loop(0, 2, half_loop, None)
            plsc.subcore_barrier()

            def co(t, _):
                r0 = sid * rows_acc + t * zrows
                pltpu.sync_copy(acc.at[pl.ds(r0, zrows)],
                                out_hbm.at[qq, pl.ds(r0, zrows)])
                return _
            lax.fori_loop(0, nz, co, None)

    return agg_kernel


def _mm1_body(x_ref, w_ref, dga_ref, dgb_ref, hs_ref, dinv_ref):
    deg = dga_ref[...] + dgb_ref[...] + 1.0
    dinv = lax.rsqrt(jnp.maximum(deg, 1e-12))
    xw = jnp.dot(x_ref[...], w_ref[...], preferred_element_type=F32)
    hs = xw * dinv
    for q in range(4):
        hs_ref[q] = hs[:, q * 64:(q + 1) * 64]
    dinv_ref[...] = dinv


def _mm2_body(agg_ref, hs1_ref, dinv_ref, b1_ref, w2_ref, hs2_ref):
    dinv = dinv_ref[...]
    hq = [jnp.tanh(dinv * (agg_ref[q] + hs1_ref[q]) + b1_ref[q])
          for q in range(4)]
    h = jnp.concatenate(hq, axis=1)
    hw = jnp.dot(h, w2_ref[...], preferred_element_type=F32) * dinv
    for q in range(4):
        hs2_ref[q] = hw[:, q * 64:(q + 1) * 64]


def _fin_body(agg_ref, hs2_ref, dinv_ref, b2_ref, fcw_ref, fcb_ref, emb_ref,
              pred_ref):
    dinv = dinv_ref[...]
    eq = [dinv * (agg_ref[q] + hs2_ref[q]) + b2_ref[q] for q in range(4)]
    emb = jnp.concatenate(eq, axis=1)
    emb_ref[...] = emb
    pred_ref[...] = jax.nn.sigmoid(
        jnp.dot(emb, fcw_ref[...], preferred_element_type=F32) + fcb_ref[0, 0])


def kernel(x, edge_index, W1, b1, W2, b2, fcW, fcb):
    n, d = x.shape
    h = W1.shape[1]
    e = edge_index.shape[1]

    n_pad = ((n + 2047) // 2048) * 2048
    e_pad = ((e + 4095) // 4096) * 4096
    blk = 1000
    grid = (n // blk,)

    src = edge_index[0]
    dst = edge_index[1]
    pad = e_pad - e
    srcp = jnp.concatenate([src, jnp.zeros((pad,), I32)])
    dstp = jnp.concatenate([dst, jnp.full((pad,), n, I32)])
    src2d = srcp.reshape(e_pad // _B, _B)
    dst2d = dstp.reshape(e_pad // _B, _B)

    deg_call = _make_deg_kernel(n_pad, e_pad)
    agg_call = _make_agg_kernel(n, n_pad, e_pad)

    degflat = deg_call(dst2d)
    dega = degflat[:n].reshape(n, 1)
    degb = degflat[n_pad:n_pad + n].reshape(n, 1)

    hs1, dinv = pl.pallas_call(
        _mm1_body,
        grid=grid,
        in_specs=[
            pl.BlockSpec((blk, d), lambda i: (i, 0)),
            pl.BlockSpec((d, h), lambda i: (0, 0)),
            pl.BlockSpec((blk, 1), lambda i: (i, 0)),
            pl.BlockSpec((blk, 1), lambda i: (i, 0)),
        ],
        out_specs=[
            pl.BlockSpec((4, blk, 64), lambda i: (0, i, 0)),
            pl.BlockSpec((blk, 1), lambda i: (i, 0)),
        ],
        out_shape=[
            jax.ShapeDtypeStruct((4, n_pad, 64), F32),
            jax.ShapeDtypeStruct((n, 1), F32),
        ],
    )(x, W1, dega, degb)

    agg1 = agg_call(hs1, src2d, dst2d)

    hs2 = pl.pallas_call(
        _mm2_body,
        grid=grid,
        in_specs=[
            pl.BlockSpec((4, blk, 64), lambda i: (0, i, 0)),
            pl.BlockSpec((4, blk, 64), lambda i: (0, i, 0)),
            pl.BlockSpec((blk, 1), lambda i: (i, 0)),
            pl.BlockSpec((4, 1, 64), lambda i: (0, 0, 0)),
            pl.BlockSpec((h, h), lambda i: (0, 0)),
        ],
        out_specs=pl.BlockSpec((4, blk, 64), lambda i: (0, i, 0)),
        out_shape=jax.ShapeDtypeStruct((4, n_pad, 64), F32),
    )(agg1, hs1, dinv, b1.reshape(4, 1, 64), W2)

    agg2 = agg_call(hs2, src2d, dst2d)

    emb, pred = pl.pallas_call(
        _fin_body,
        grid=grid,
        in_specs=[
            pl.BlockSpec((4, blk, 64), lambda i: (0, i, 0)),
            pl.BlockSpec((4, blk, 64), lambda i: (0, i, 0)),
            pl.BlockSpec((blk, 1), lambda i: (i, 0)),
            pl.BlockSpec((4, 1, 64), lambda i: (0, 0, 0)),
            pl.BlockSpec((h, 1), lambda i: (0, 0)),
            pl.BlockSpec((1, 1), lambda i: (0, 0)),
        ],
        out_specs=[
            pl.BlockSpec((blk, h), lambda i: (i, 0)),
            pl.BlockSpec((blk, 1), lambda i: (i, 0)),
        ],
        out_shape=[
            jax.ShapeDtypeStruct((n, h), F32),
            jax.ShapeDtypeStruct((n, 1), F32),
        ],
    )(agg2, hs2, dinv, b2.reshape(4, 1, 64), fcW, fcb.reshape(1, 1))

    return (emb, pred)


# speedup vs baseline: 1.2188x; 1.2188x over previous
import functools

import jax
import jax.numpy as jnp
from jax import lax
from jax.experimental import pallas as pl
from jax.experimental.pallas import tpu as pltpu
from jax.experimental.pallas import tpu_sc as plsc

F32 = jnp.float32
I32 = jnp.int32

_NS = 16
_NC = 2
_B = 128


def _sc_mesh():
    return plsc.VectorSubcoreMesh(core_axis_name="c", subcore_axis_name="s")


def _make_deg_kernel(n_pad, e_pad):
    rows_tile = n_pad // _NS
    nb = e_pad // (_NC * _NS * _B)

    @functools.partial(
        pl.kernel,
        out_type=jax.ShapeDtypeStruct((_NC * n_pad,), F32),
        mesh=_sc_mesh(),
        scratch_types=[
            pltpu.VMEM((nb, _B), I32),
            pltpu.VMEM((_B,), F32),
            pltpu.VMEM((rows_tile,), F32),
            pltpu.VMEM_SHARED((n_pad,), F32),
        ],
    )
    def deg_kernel(dst_hbm, out_hbm, dstv, ones, zbuf, acc):
        cid = lax.axis_index("c")
        sid = lax.axis_index("s")
        wid = cid * _NS + sid

        def fill_ones(i, _):
            ones[pl.ds(i * 16, 16)] = jnp.ones((16,), F32)
            return _
        lax.fori_loop(0, _B // 16, fill_ones, None)

        def fill_z(i, _):
            zbuf[pl.ds(i * 16, 16)] = jnp.zeros((16,), F32)
            return _
        lax.fori_loop(0, rows_tile // 16, fill_z, None)
        pltpu.sync_copy(zbuf, acc.at[pl.ds(sid * rows_tile, rows_tile)])
        plsc.subcore_barrier()

        pltpu.sync_copy(dst_hbm.at[pl.ds(wid * nb, nb)], dstv)

        def scat(j, _):
            pltpu.sync_copy(ones, acc.at[dstv.at[j]], add=True)
            return _
        lax.fori_loop(0, nb, scat, None)
        plsc.subcore_barrier()

        off = cid * n_pad + sid * rows_tile
        pltpu.sync_copy(acc.at[pl.ds(sid * rows_tile, rows_tile)],
                        out_hbm.at[pl.ds(off, rows_tile)])

    return deg_kernel


def _make_agg_kernel(n, n_pad, e_pad):
    rows_acc = n_pad // _NS
    nb = e_pad // (_NS * _B)
    nh = nb // 2
    zrows = 128
    nz = rows_acc // zrows

    @functools.partial(
        pl.kernel,
        out_type=jax.ShapeDtypeStruct((_NC, n_pad, 128), F32),
        mesh=_sc_mesh(),
        compiler_params=pltpu.CompilerParams(use_tc_tiling_on_sc=False),
        scratch_types=[
            pltpu.VMEM((nh, _B), I32),
            pltpu.VMEM((nh, _B), I32),
            pltpu.VMEM((_B, 64), F32),
            pltpu.VMEM((_B, 64), F32),
            pltpu.VMEM((_B, 64), F32),
            pltpu.VMEM((_B, 64), F32),
            pltpu.VMEM_SHARED((n_pad, 64), F32),
            pltpu.VMEM_SHARED((n_pad, 64), F32),
            [pltpu.SemaphoreType.DMA] * 4,
            [pltpu.SemaphoreType.DMA] * 4,
        ],
    )
    def agg_kernel(hs_hbm, src_hbm, dst_hbm, out_hbm, srcv, dstv, rows0,
                   rows1, rows2, rows3, table, acc, sg, ss):
        cid = lax.axis_index("c")
        sid = lax.axis_index("s")

        for q in range(2):
            pltpu.sync_copy(
                hs_hbm.at[cid, pl.ds(sid * rows_acc, rows_acc),
                          pl.ds(q * 64, 64)],
                table.at[pl.ds(sid * rows_acc, rows_acc)])

            def fz(k, _):
                j = k // 4
                i = k - j * 4
                rows0[j, pl.ds(i * 16, 16)] = jnp.zeros((16,), F32)
                return _
            lax.fori_loop(0, zrows * 4, fz, None)

            def zc(t, _):
                pltpu.sync_copy(rows0,
                                acc.at[pl.ds(sid * rows_acc + t * zrows, zrows)])
                return _
            lax.fori_loop(0, nz, zc, None)
            plsc.subcore_barrier()

            bufs = (rows0, rows1, rows2, rows3)

            def half_loop(hf, _):
                base = sid * nb + hf * nh
                pltpu.sync_copy(src_hbm.at[pl.ds(base, nh)], srcv)
                pltpu.sync_copy(dst_hbm.at[pl.ds(base, nh)], dstv)

                pltpu.async_copy(table.at[srcv.at[0]], rows0, sg[0])
                pltpu.async_copy(table.at[srcv.at[1]], rows1, sg[1])

                def quad(g, __):
                    for p in range(4):
                        j = 4 * g + p
                        q = (p + 2) % 4
                        pltpu.make_async_copy(
                            table.at[srcv.at[j]], bufs[p], sg[p]).wait()
                        pltpu.async_copy(
                            bufs[p], acc.at[dstv.at[j]], ss[p], add=True)

                        @pl.when(j >= 2)
                        def _wait_prev():
                            pltpu.make_async_copy(
                                bufs[q], acc.at[dstv.at[j]], ss[q]).wait()

                        @pl.when(j + 2 < nh)
                        def _issue_next():
                            pltpu.async_copy(
                                table.at[srcv.at[j + 2]], bufs[q], sg[q])
                    return __
                lax.fori_loop(0, nh // 4, quad, None)
                pltpu.make_async_copy(
                    bufs[(nh - 2) % 4], acc.at[dstv.at[nh - 2]], ss[(nh - 2) % 4]).wait()
                pltpu.make_async_copy(
                    bufs[(nh - 1) % 4], acc.at[dstv.at[nh - 1]], ss[(nh - 1) % 4]).wait()
                return _
            lax.fori_loop(0, 2, half_loop, None)
            plsc.subcore_barrier()

            def co(t, _):
                r0 = sid * rows_acc + t * zrows
                pltpu.sync_copy(acc.at[pl.ds(r0, zrows)],
                                out_hbm.at[cid, pl.ds(r0, zrows),
                                           pl.ds(q * 64, 64)])
                return _
            lax.fori_loop(0, nz, co, None)

    return agg_kernel


def _mm1_body(x_ref, w_ref, dga_ref, dgb_ref, hs_ref, dinv_ref):
    deg = dga_ref[...] + dgb_ref[...] + 1.0
    dinv = lax.rsqrt(jnp.maximum(deg, 1e-12))
    xw = jnp.dot(x_ref[...], w_ref[...], preferred_element_type=F32)
    hs = xw * dinv
    hs_ref[0] = hs[:, :128]
    hs_ref[1] = hs[:, 128:]
    dinv_ref[...] = dinv


def _mm2_body(agg_ref, hs1_ref, dinv_ref, b1_ref, w2_ref, hs2_ref):
    dinv = dinv_ref[...]
    hq = [jnp.tanh(dinv * (agg_ref[q] + hs1_ref[q]) + b1_ref[q])
          for q in range(2)]
    h = jnp.concatenate(hq, axis=1)
    hw = jnp.dot(h, w2_ref[...], preferred_element_type=F32) * dinv
    hs2_ref[0] = hw[:, :128]
    hs2_ref[1] = hw[:, 128:]


def _fin_body(agg_ref, hs2_ref, dinv_ref, b2_ref, fcw_ref, fcb_ref, emb_ref,
              pred_ref):
    dinv = dinv_ref[...]
    eq = [dinv * (agg_ref[q] + hs2_ref[q]) + b2_ref[q] for q in range(2)]
    emb = jnp.concatenate(eq, axis=1)
    emb_ref[...] = emb
    pred_ref[...] = jax.nn.sigmoid(
        jnp.dot(emb, fcw_ref[...], preferred_element_type=F32) + fcb_ref[0, 0])


def kernel(x, edge_index, W1, b1, W2, b2, fcW, fcb):
    n, d = x.shape
    h = W1.shape[1]
    e = edge_index.shape[1]

    n_pad = ((n + 2047) // 2048) * 2048
    e_pad = ((e + 4095) // 4096) * 4096
    blk = 1000
    grid = (n // blk,)

    src = edge_index[0]
    dst = edge_index[1]
    pad = e_pad - e
    srcp = jnp.concatenate([src, jnp.zeros((pad,), I32)])
    dstp = jnp.concatenate([dst, jnp.full((pad,), n, I32)])
    src2d = srcp.reshape(e_pad // _B, _B)
    dst2d = dstp.reshape(e_pad // _B, _B)

    deg_call = _make_deg_kernel(n_pad, e_pad)
    agg_call = _make_agg_kernel(n, n_pad, e_pad)

    degflat = deg_call(dst2d)
    dega = degflat[:n].reshape(n, 1)
    degb = degflat[n_pad:n_pad + n].reshape(n, 1)

    hs1, dinv = pl.pallas_call(
        _mm1_body,
        grid=grid,
        in_specs=[
            pl.BlockSpec((blk, d), lambda i: (i, 0)),
            pl.BlockSpec((d, h), lambda i: (0, 0)),
            pl.BlockSpec((blk, 1), lambda i: (i, 0)),
            pl.BlockSpec((blk, 1), lambda i: (i, 0)),
        ],
        out_specs=[
            pl.BlockSpec((2, blk, 128), lambda i: (0, i, 0)),
            pl.BlockSpec((blk, 1), lambda i: (i, 0)),
        ],
        out_shape=[
            jax.ShapeDtypeStruct((2, n_pad, 128), F32),
            jax.ShapeDtypeStruct((n, 1), F32),
        ],
    )(x, W1, dega, degb)

    agg1 = agg_call(hs1, src2d, dst2d)

    hs2 = pl.pallas_call(
        _mm2_body,
        grid=grid,
        in_specs=[
            pl.BlockSpec((2, blk, 128), lambda i: (0, i, 0)),
            pl.BlockSpec((2, blk, 128), lambda i: (0, i, 0)),
            pl.BlockSpec((blk, 1), lambda i: (i, 0)),
            pl.BlockSpec((2, 1, 128), lambda i: (0, 0, 0)),
            pl.BlockSpec((h, h), lambda i: (0, 0)),
        ],
        out_specs=pl.BlockSpec((2, blk, 128), lambda i: (0, i, 0)),
        out_shape=jax.ShapeDtypeStruct((2, n_pad, 128), F32),
    )(agg1, hs1, dinv, b1.reshape(2, 1, 128), W2)

    agg2 = agg_call(hs2, src2d, dst2d)

    emb, pred = pl.pallas_call(
        _fin_body,
        grid=grid,
        in_specs=[
            pl.BlockSpec((2, blk, 128), lambda i: (0, i, 0)),
            pl.BlockSpec((2, blk, 128), lambda i: (0, i, 0)),
            pl.BlockSpec((blk, 1), lambda i: (i, 0)),
            pl.BlockSpec((2, 1, 128), lambda i: (0, 0, 0)),
            pl.BlockSpec((h, 1), lambda i: (0, 0)),
            pl.BlockSpec((1, 1), lambda i: (0, 0)),
        ],
        out_specs=[
            pl.BlockSpec((blk, h), lambda i: (i, 0)),
            pl.BlockSpec((blk, 1), lambda i: (i, 0)),
        ],
        out_shape=[
            jax.ShapeDtypeStruct((n, h), F32),
            jax.ShapeDtypeStruct((n, 1), F32),
        ],
    )(agg2, hs2, dinv, b2.reshape(2, 1, 128), fcW, fcb.reshape(1, 1))

    return (emb, pred)


# speedup vs baseline: 1.2282x; 1.0077x over previous
import functools

import jax
import jax.numpy as jnp
from jax import lax
from jax.experimental import pallas as pl
from jax.experimental.pallas import tpu as pltpu
from jax.experimental.pallas import tpu_sc as plsc

F32 = jnp.float32
I32 = jnp.int32

_NS = 16
_NC = 2
_B = 128


def _sc_mesh():
    return plsc.VectorSubcoreMesh(core_axis_name="c", subcore_axis_name="s")


def _make_deg_kernel(n_pad, e_pad):
    rows_tile = n_pad // _NS
    nb = e_pad // (_NC * _NS * _B)

    @functools.partial(
        pl.kernel,
        out_type=jax.ShapeDtypeStruct((_NC * n_pad,), F32),
        mesh=_sc_mesh(),
        scratch_types=[
            pltpu.VMEM((nb, _B), I32),
            pltpu.VMEM((_B,), F32),
            pltpu.VMEM((rows_tile,), F32),
            pltpu.VMEM_SHARED((n_pad,), F32),
        ],
    )
    def deg_kernel(dst_hbm, out_hbm, dstv, ones, zbuf, acc):
        cid = lax.axis_index("c")
        sid = lax.axis_index("s")
        wid = cid * _NS + sid

        def fill_ones(i, _):
            ones[pl.ds(i * 16, 16)] = jnp.ones((16,), F32)
            return _
        lax.fori_loop(0, _B // 16, fill_ones, None)

        def fill_z(i, _):
            zbuf[pl.ds(i * 16, 16)] = jnp.zeros((16,), F32)
            return _
        lax.fori_loop(0, rows_tile // 16, fill_z, None)
        pltpu.sync_copy(zbuf, acc.at[pl.ds(sid * rows_tile, rows_tile)])
        plsc.subcore_barrier()

        pltpu.sync_copy(dst_hbm.at[pl.ds(wid * nb, nb)], dstv)

        def scat(j, _):
            pltpu.sync_copy(ones, acc.at[dstv.at[j]], add=True)
            return _
        lax.fori_loop(0, nb, scat, None)
        plsc.subcore_barrier()

        off = cid * n_pad + sid * rows_tile
        pltpu.sync_copy(acc.at[pl.ds(sid * rows_tile, rows_tile)],
                        out_hbm.at[pl.ds(off, rows_tile)])

    return deg_kernel


def _make_agg_kernel(n, n_pad, e_pad):
    rows_acc = n_pad // _NS
    nb = e_pad // (_NS * _B)
    nh = nb // 2
    zrows = 128
    nz = rows_acc // zrows

    @functools.partial(
        pl.kernel,
        out_type=jax.ShapeDtypeStruct((_NC, n_pad, 128), F32),
        mesh=_sc_mesh(),
        compiler_params=pltpu.CompilerParams(use_tc_tiling_on_sc=False),
        scratch_types=[
            pltpu.VMEM((nh, _B), I32),
            pltpu.VMEM((nh, _B), I32),
            pltpu.VMEM((_B, 64), F32),
            pltpu.VMEM((_B, 64), F32),
            pltpu.VMEM((_B, 64), F32),
            pltpu.VMEM((_B, 64), F32),
            pltpu.VMEM_SHARED((n_pad, 64), F32),
            pltpu.VMEM_SHARED((n_pad, 64), F32),
            [pltpu.SemaphoreType.DMA] * 4,
            [pltpu.SemaphoreType.DMA] * 4,
        ],
    )
    def agg_kernel(hs_hbm, src_hbm, dst_hbm, out_hbm, srcv, dstv, rows0,
                   rows1, rows2, rows3, table, acc, sg, ss):
        cid = lax.axis_index("c")
        sid = lax.axis_index("s")

        for q in range(2):
            pltpu.sync_copy(
                hs_hbm.at[cid, pl.ds(sid * rows_acc, rows_acc),
                          pl.ds(q * 64, 64)],
                table.at[pl.ds(sid * rows_acc, rows_acc)])
            pltpu.sync_copy(
                hs_hbm.at[cid, pl.ds(sid * rows_acc, rows_acc),
                          pl.ds(q * 64, 64)],
                acc.at[pl.ds(sid * rows_acc, rows_acc)])
            plsc.subcore_barrier()

            bufs = (rows0, rows1, rows2, rows3)

            def half_loop(hf, _):
                base = sid * nb + hf * nh
                pltpu.sync_copy(src_hbm.at[pl.ds(base, nh)], srcv)
                pltpu.sync_copy(dst_hbm.at[pl.ds(base, nh)], dstv)

                pltpu.async_copy(table.at[srcv.at[0]], rows0, sg[0])
                pltpu.async_copy(table.at[srcv.at[1]], rows1, sg[1])

                def quad(g, __):
                    for p in range(4):
                        j = 4 * g + p
                        q = (p + 2) % 4
                        pltpu.make_async_copy(
                            table.at[srcv.at[j]], bufs[p], sg[p]).wait()
                        pltpu.async_copy(
                            bufs[p], acc.at[dstv.at[j]], ss[p], add=True)

                        @pl.when(j >= 2)
                        def _wait_prev():
                            pltpu.make_async_copy(
                                bufs[q], acc.at[dstv.at[j]], ss[q]).wait()

                        @pl.when(j + 2 < nh)
                        def _issue_next():
                            pltpu.async_copy(
                                table.at[srcv.at[j + 2]], bufs[q], sg[q])
                    return __
                lax.fori_loop(0, nh // 4, quad, None)
                pltpu.make_async_copy(
                    bufs[(nh - 2) % 4], acc.at[dstv.at[nh - 2]], ss[(nh - 2) % 4]).wait()
                pltpu.make_async_copy(
                    bufs[(nh - 1) % 4], acc.at[dstv.at[nh - 1]], ss[(nh - 1) % 4]).wait()
                return _
            lax.fori_loop(0, 2, half_loop, None)
            plsc.subcore_barrier()

            def co(t, _):
                r0 = sid * rows_acc + t * zrows
                pltpu.sync_copy(acc.at[pl.ds(r0, zrows)],
                                out_hbm.at[cid, pl.ds(r0, zrows),
                                           pl.ds(q * 64, 64)])
                return _
            lax.fori_loop(0, nz, co, None)

    return agg_kernel


def _mm1_body(x_ref, w_ref, dga_ref, dgb_ref, hs_ref, dinv_ref):
    deg = dga_ref[...] + dgb_ref[...] + 1.0
    dinv = lax.rsqrt(jnp.maximum(deg, 1e-12))
    xw = jnp.dot(x_ref[...], w_ref[...], preferred_element_type=F32)
    hs = xw * dinv
    hs_ref[0] = hs[:, :128]
    hs_ref[1] = hs[:, 128:]
    dinv_ref[...] = dinv


def _mm2_body(agg_ref, dinv_ref, b1_ref, w2_ref, hs2_ref):
    dinv = dinv_ref[...]
    hq = [jnp.tanh(dinv * agg_ref[q] + b1_ref[q]) for q in range(2)]
    h = jnp.concatenate(hq, axis=1)
    hw = jnp.dot(h, w2_ref[...], preferred_element_type=F32) * dinv
    hs2_ref[0] = hw[:, :128]
    hs2_ref[1] = hw[:, 128:]


def _fin_body(agg_ref, dinv_ref, b2_ref, fcw_ref, fcb_ref, emb_ref,
              pred_ref):
    dinv = dinv_ref[...]
    eq = [dinv * agg_ref[q] + b2_ref[q] for q in range(2)]
    emb = jnp.concatenate(eq, axis=1)
    emb_ref[...] = emb
    pred_ref[...] = jax.nn.sigmoid(
        jnp.dot(emb, fcw_ref[...], preferred_element_type=F32) + fcb_ref[0, 0])


def kernel(x, edge_index, W1, b1, W2, b2, fcW, fcb):
    n, d = x.shape
    h = W1.shape[1]
    e = edge_index.shape[1]

    n_pad = ((n + 2047) // 2048) * 2048
    e_pad = ((e + 4095) // 4096) * 4096
    blk = 1000
    grid = (n // blk,)

    src = edge_index[0]
    dst = edge_index[1]
    pad = e_pad - e
    srcp = jnp.concatenate([src, jnp.zeros((pad,), I32)])
    dstp = jnp.concatenate([dst, jnp.full((pad,), n, I32)])
    src2d = srcp.reshape(e_pad // _B, _B)
    dst2d = dstp.reshape(e_pad // _B, _B)

    deg_call = _make_deg_kernel(n_pad, e_pad)
    agg_call = _make_agg_kernel(n, n_pad, e_pad)

    degflat = deg_call(dst2d)
    dega = degflat[:n].reshape(n, 1)
    degb = degflat[n_pad:n_pad + n].reshape(n, 1)

    hs1, dinv = pl.pallas_call(
        _mm1_body,
        grid=grid,
        in_specs=[
            pl.BlockSpec((blk, d), lambda i: (i, 0)),
            pl.BlockSpec((d, h), lambda i: (0, 0)),
            pl.BlockSpec((blk, 1), lambda i: (i, 0)),
            pl.BlockSpec((blk, 1), lambda i: (i, 0)),
        ],
        out_specs=[
            pl.BlockSpec((2, blk, 128), lambda i: (0, i, 0)),
            pl.BlockSpec((blk, 1), lambda i: (i, 0)),
        ],
        out_shape=[
            jax.ShapeDtypeStruct((2, n_pad, 128), F32),
            jax.ShapeDtypeStruct((n, 1), F32),
        ],
    )(x, W1, dega, degb)

    agg1 = agg_call(hs1, src2d, dst2d)

    hs2 = pl.pallas_call(
        _mm2_body,
        grid=grid,
        in_specs=[
            pl.BlockSpec((2, blk, 128), lambda i: (0, i, 0)),
            pl.BlockSpec((blk, 1), lambda i: (i, 0)),
            pl.BlockSpec((2, 1, 128), lambda i: (0, 0, 0)),
            pl.BlockSpec((h, h), lambda i: (0, 0)),
        ],
        out_specs=pl.BlockSpec((2, blk, 128), lambda i: (0, i, 0)),
        out_shape=jax.ShapeDtypeStruct((2, n_pad, 128), F32),
    )(agg1, dinv, b1.reshape(2, 1, 128), W2)

    agg2 = agg_call(hs2, src2d, dst2d)

    emb, pred = pl.pallas_call(
        _fin_body,
        grid=grid,
        in_specs=[
            pl.BlockSpec((2, blk, 128), lambda i: (0, i, 0)),
            pl.BlockSpec((blk, 1), lambda i: (i, 0)),
            pl.BlockSpec((2, 1, 128), lambda i: (0, 0, 0)),
            pl.BlockSpec((h, 1), lambda i: (0, 0)),
            pl.BlockSpec((1, 1), lambda i: (0, 0)),
        ],
        out_specs=[
            pl.BlockSpec((blk, h), lambda i: (i, 0)),
            pl.BlockSpec((blk, 1), lambda i: (i, 0)),
        ],
        out_shape=[
            jax.ShapeDtypeStruct((n, h), F32),
            jax.ShapeDtypeStruct((n, 1), F32),
        ],
    )(agg2, dinv, b2.reshape(2, 1, 128), fcW, fcb.reshape(1, 1))

    return (emb, pred)


# speedup vs baseline: 1.2532x; 1.0204x over previous
import functools

import jax
import jax.numpy as jnp
from jax import lax
from jax.experimental import pallas as pl
from jax.experimental.pallas import tpu as pltpu
from jax.experimental.pallas import tpu_sc as plsc

F32 = jnp.float32
I32 = jnp.int32

_NS = 16
_NC = 2
_B = 128
_BA = 112


def _sc_mesh():
    return plsc.VectorSubcoreMesh(core_axis_name="c", subcore_axis_name="s")


def _make_deg_kernel(n_pad, e_pad):
    rows_tile = n_pad // _NS
    nb = e_pad // (_NC * _NS * _B)

    @functools.partial(
        pl.kernel,
        out_type=jax.ShapeDtypeStruct((_NC * n_pad,), F32),
        mesh=_sc_mesh(),
        scratch_types=[
            pltpu.VMEM((nb, _B), I32),
            pltpu.VMEM((_B,), F32),
            pltpu.VMEM((rows_tile,), F32),
            pltpu.VMEM_SHARED((n_pad,), F32),
        ],
    )
    def deg_kernel(dst_hbm, out_hbm, dstv, ones, zbuf, acc):
        cid = lax.axis_index("c")
        sid = lax.axis_index("s")
        wid = cid * _NS + sid

        def fill_ones(i, _):
            ones[pl.ds(i * 16, 16)] = jnp.ones((16,), F32)
            return _
        lax.fori_loop(0, _B // 16, fill_ones, None)

        def fill_z(i, _):
            zbuf[pl.ds(i * 16, 16)] = jnp.zeros((16,), F32)
            return _
        lax.fori_loop(0, rows_tile // 16, fill_z, None)
        pltpu.sync_copy(zbuf, acc.at[pl.ds(sid * rows_tile, rows_tile)])
        plsc.subcore_barrier()

        pltpu.sync_copy(dst_hbm.at[pl.ds(wid * nb, nb)], dstv)

        def scat(j, _):
            pltpu.sync_copy(ones, acc.at[dstv.at[j]], add=True)
            return _
        lax.fori_loop(0, nb, scat, None)
        plsc.subcore_barrier()

        off = cid * n_pad + sid * rows_tile
        pltpu.sync_copy(acc.at[pl.ds(sid * rows_tile, rows_tile)],
                        out_hbm.at[pl.ds(off, rows_tile)])

    return deg_kernel


def _make_agg_kernel(n, n_pad, e_pad):
    rows_acc = n_pad // _NS
    nb = e_pad // (_NS * _BA)
    nh = nb // 2
    zrows = 128
    nz = rows_acc // zrows

    @functools.partial(
        pl.kernel,
        out_type=jax.ShapeDtypeStruct((_NC, n_pad, 128), F32),
        mesh=_sc_mesh(),
        compiler_params=pltpu.CompilerParams(use_tc_tiling_on_sc=False),
        scratch_types=[
            pltpu.VMEM((nh, _BA), I32),
            pltpu.VMEM((nh, _BA), I32),
            [pltpu.VMEM((_BA, 64), F32)] * 5,
            pltpu.VMEM_SHARED((n_pad, 64), F32),
            pltpu.VMEM_SHARED((n_pad, 64), F32),
            [pltpu.SemaphoreType.DMA] * 5,
            [pltpu.SemaphoreType.DMA] * 5,
        ],
    )
    def agg_kernel(hs_hbm, src_hbm, dst_hbm, out_hbm, srcv, dstv, bufs,
                   table, acc, sg, ss):
        cid = lax.axis_index("c")
        sid = lax.axis_index("s")

        for q in range(2):
            pltpu.sync_copy(
                hs_hbm.at[cid, pl.ds(sid * rows_acc, rows_acc),
                          pl.ds(q * 64, 64)],
                table.at[pl.ds(sid * rows_acc, rows_acc)])
            pltpu.sync_copy(
                hs_hbm.at[cid, pl.ds(sid * rows_acc, rows_acc),
                          pl.ds(q * 64, 64)],
                acc.at[pl.ds(sid * rows_acc, rows_acc)])
            plsc.subcore_barrier()

            def half_loop(hf, _):
                base = sid * nb + hf * nh
                pltpu.sync_copy(src_hbm.at[pl.ds(base, nh)], srcv)
                pltpu.sync_copy(dst_hbm.at[pl.ds(base, nh)], dstv)

                for p in range(3):
                    pltpu.async_copy(table.at[srcv.at[p]], bufs[p], sg[p])

                def quint(g, __):
                    for p in range(5):
                        j = 5 * g + p
                        q = (p + 3) % 5
                        pltpu.make_async_copy(
                            table.at[srcv.at[j]], bufs[p], sg[p]).wait()
                        pltpu.async_copy(
                            bufs[p], acc.at[dstv.at[j]], ss[p], add=True)

                        @pl.when(j >= 2)
                        def _wait_prev():
                            pltpu.make_async_copy(
                                bufs[q], acc.at[dstv.at[j]], ss[q]).wait()

                        @pl.when(j + 3 < nh)
                        def _issue_next():
                            pltpu.async_copy(
                                table.at[srcv.at[j + 3]], bufs[q], sg[q])
                    return __
                lax.fori_loop(0, nh // 5, quint, None)
                pltpu.make_async_copy(
                    bufs[(nh - 2) % 5], acc.at[dstv.at[nh - 2]], ss[(nh - 2) % 5]).wait()
                pltpu.make_async_copy(
                    bufs[(nh - 1) % 5], acc.at[dstv.at[nh - 1]], ss[(nh - 1) % 5]).wait()
                return _
            lax.fori_loop(0, 2, half_loop, None)
            plsc.subcore_barrier()

            def co(t, _):
                r0 = sid * rows_acc + t * zrows
                pltpu.sync_copy(acc.at[pl.ds(r0, zrows)],
                                out_hbm.at[cid, pl.ds(r0, zrows),
                                           pl.ds(q * 64, 64)])
                return _
            lax.fori_loop(0, nz, co, None)

    return agg_kernel


def _mm1_body(x_ref, w_ref, dga_ref, dgb_ref, hs_ref, dinv_ref):
    deg = dga_ref[...] + dgb_ref[...] + 1.0
    dinv = lax.rsqrt(jnp.maximum(deg, 1e-12))
    xw = jnp.dot(x_ref[...], w_ref[...], preferred_element_type=F32)
    hs = xw * dinv
    hs_ref[0] = hs[:, :128]
    hs_ref[1] = hs[:, 128:]
    dinv_ref[...] = dinv


def _mm2_body(agg_ref, dinv_ref, b1_ref, w2_ref, hs2_ref):
    dinv = dinv_ref[...]
    hq = [jnp.tanh(dinv * agg_ref[q] + b1_ref[q]) for q in range(2)]
    h = jnp.concatenate(hq, axis=1)
    hw = jnp.dot(h, w2_ref[...], preferred_element_type=F32) * dinv
    hs2_ref[0] = hw[:, :128]
    hs2_ref[1] = hw[:, 128:]


def _fin_body(agg_ref, dinv_ref, b2_ref, fcw_ref, fcb_ref, emb_ref,
              pred_ref):
    dinv = dinv_ref[...]
    eq = [dinv * agg_ref[q] + b2_ref[q] for q in range(2)]
    emb = jnp.concatenate(eq, axis=1)
    emb_ref[...] = emb
    pred_ref[...] = jax.nn.sigmoid(
        jnp.dot(emb, fcw_ref[...], preferred_element_type=F32) + fcb_ref[0, 0])


def kernel(x, edge_index, W1, b1, W2, b2, fcW, fcb):
    n, d = x.shape
    h = W1.shape[1]
    e = edge_index.shape[1]

    n_pad = ((n + 2047) // 2048) * 2048
    e_pad = ((e + 4095) // 4096) * 4096
    ea_q = _NS * _BA * 10
    e_pad_a = ((e + ea_q - 1) // ea_q) * ea_q
    blk = 1000
    grid = (n // blk,)

    src = edge_index[0]
    dst = edge_index[1]
    dstp = jnp.concatenate([dst, jnp.full((e_pad - e,), n, I32)])
    dst2d = dstp.reshape(e_pad // _B, _B)
    srcpa = jnp.concatenate([src, jnp.zeros((e_pad_a - e,), I32)])
    dstpa = jnp.concatenate([dst, jnp.full((e_pad_a - e,), n, I32)])
    src2da = srcpa.reshape(e_pad_a // _BA, _BA)
    dst2da = dstpa.reshape(e_pad_a // _BA, _BA)

    deg_call = _make_deg_kernel(n_pad, e_pad)
    agg_call = _make_agg_kernel(n, n_pad, e_pad_a)

    degflat = deg_call(dst2d)
    dega = degflat[:n].reshape(n, 1)
    degb = degflat[n_pad:n_pad + n].reshape(n, 1)

    hs1, dinv = pl.pallas_call(
        _mm1_body,
        grid=grid,
        in_specs=[
            pl.BlockSpec((blk, d), lambda i: (i, 0)),
            pl.BlockSpec((d, h), lambda i: (0, 0)),
            pl.BlockSpec((blk, 1), lambda i: (i, 0)),
            pl.BlockSpec((blk, 1), lambda i: (i, 0)),
        ],
        out_specs=[
            pl.BlockSpec((2, blk, 128), lambda i: (0, i, 0)),
            pl.BlockSpec((blk, 1), lambda i: (i, 0)),
        ],
        out_shape=[
            jax.ShapeDtypeStruct((2, n_pad, 128), F32),
            jax.ShapeDtypeStruct((n, 1), F32),
        ],
    )(x, W1, dega, degb)

    agg1 = agg_call(hs1, src2da, dst2da)

    hs2 = pl.pallas_call(
        _mm2_body,
        grid=grid,
        in_specs=[
            pl.BlockSpec((2, blk, 128), lambda i: (0, i, 0)),
            pl.BlockSpec((blk, 1), lambda i: (i, 0)),
            pl.BlockSpec((2, 1, 128), lambda i: (0, 0, 0)),
            pl.BlockSpec((h, h), lambda i: (0, 0)),
        ],
        out_specs=pl.BlockSpec((2, blk, 128), lambda i: (0, i, 0)),
        out_shape=jax.ShapeDtypeStruct((2, n_pad, 128), F32),
    )(agg1, dinv, b1.reshape(2, 1, 128), W2)

    agg2 = agg_call(hs2, src2da, dst2da)

    emb, pred = pl.pallas_call(
        _fin_body,
        grid=grid,
        in_specs=[
            pl.BlockSpec((2, blk, 128), lambda i: (0, i, 0)),
            pl.BlockSpec((blk, 1), lambda i: (i, 0)),
            pl.BlockSpec((2, 1, 128), lambda i: (0, 0, 0)),
            pl.BlockSpec((h, 1), lambda i: (0, 0)),
            pl.BlockSpec((1, 1), lambda i: (0, 0)),
        ],
        out_specs=[
            pl.BlockSpec((blk, h), lambda i: (i, 0)),
            pl.BlockSpec((blk, 1), lambda i: (i, 0)),
        ],
        out_shape=[
            jax.ShapeDtypeStruct((n, h), F32),
            jax.ShapeDtypeStruct((n, 1), F32),
        ],
    )(agg2, dinv, b2.reshape(2, 1, 128), fcW, fcb.reshape(1, 1))

    return (emb, pred)


# speedup vs baseline: 1.2995x; 1.0369x over previous
import functools

import jax
import jax.numpy as jnp
from jax import lax
from jax.experimental import pallas as pl
from jax.experimental.pallas import tpu as pltpu
from jax.experimental.pallas import tpu_sc as plsc

F32 = jnp.float32
I32 = jnp.int32

_NS = 16
_NC = 2
_B = 128
_BA = 112


def _sc_mesh():
    return plsc.VectorSubcoreMesh(core_axis_name="c", subcore_axis_name="s")


def _make_deg_kernel(n_pad, e_pad):
    rows_tile = n_pad // _NS
    nb = e_pad // (_NC * _NS * _BA)

    @functools.partial(
        pl.kernel,
        out_type=jax.ShapeDtypeStruct((_NC * n_pad,), F32),
        mesh=_sc_mesh(),
        compiler_params=pltpu.CompilerParams(use_tc_tiling_on_sc=False),
        scratch_types=[
            pltpu.VMEM((nb, _BA), I32),
            pltpu.VMEM((_BA,), F32),
            pltpu.VMEM((rows_tile,), F32),
            pltpu.VMEM_SHARED((n_pad,), F32),
        ],
    )
    def deg_kernel(dst_hbm, out_hbm, dstv, ones, zbuf, acc):
        cid = lax.axis_index("c")
        sid = lax.axis_index("s")
        wid = cid * _NS + sid

        def fill_ones(i, _):
            ones[pl.ds(i * 16, 16)] = jnp.ones((16,), F32)
            return _
        lax.fori_loop(0, _BA // 16, fill_ones, None)

        def fill_z(i, _):
            zbuf[pl.ds(i * 16, 16)] = jnp.zeros((16,), F32)
            return _
        lax.fori_loop(0, rows_tile // 16, fill_z, None)
        pltpu.sync_copy(zbuf, acc.at[pl.ds(sid * rows_tile, rows_tile)])
        plsc.subcore_barrier()

        pltpu.sync_copy(dst_hbm.at[pl.ds(wid * nb, nb)], dstv)

        def scat(j, _):
            pltpu.sync_copy(ones, acc.at[dstv.at[j]], add=True)
            return _
        lax.fori_loop(0, nb, scat, None)
        plsc.subcore_barrier()

        off = cid * n_pad + sid * rows_tile
        pltpu.sync_copy(acc.at[pl.ds(sid * rows_tile, rows_tile)],
                        out_hbm.at[pl.ds(off, rows_tile)])

    return deg_kernel


def _make_agg_kernel(n, n_pad, e_pad):
    rows_acc = n_pad // _NS
    nb = e_pad // (_NS * _BA)
    nh = nb // 2
    zrows = 128
    nz = rows_acc // zrows

    @functools.partial(
        pl.kernel,
        out_type=jax.ShapeDtypeStruct((_NC, n_pad, 128), F32),
        mesh=_sc_mesh(),
        compiler_params=pltpu.CompilerParams(use_tc_tiling_on_sc=False),
        scratch_types=[
            pltpu.VMEM((nh, _BA), I32),
            pltpu.VMEM((nh, _BA), I32),
            [pltpu.VMEM((_BA, 64), F32)] * 5,
            pltpu.VMEM_SHARED((n_pad, 64), F32),
            pltpu.VMEM_SHARED((n_pad, 64), F32),
            [pltpu.SemaphoreType.DMA] * 5,
            [pltpu.SemaphoreType.DMA] * 5,
        ],
    )
    def agg_kernel(hs_hbm, src_hbm, dst_hbm, out_hbm, srcv, dstv, bufs,
                   table, acc, sg, ss):
        cid = lax.axis_index("c")
        sid = lax.axis_index("s")

        for q in range(2):
            pltpu.sync_copy(
                hs_hbm.at[cid, pl.ds(sid * rows_acc, rows_acc),
                          pl.ds(q * 64, 64)],
                table.at[pl.ds(sid * rows_acc, rows_acc)])
            pltpu.sync_copy(
                hs_hbm.at[cid, pl.ds(sid * rows_acc, rows_acc),
                          pl.ds(q * 64, 64)],
                acc.at[pl.ds(sid * rows_acc, rows_acc)])
            plsc.subcore_barrier()

            def half_loop(hf, _):
                base = sid * nb + hf * nh
                pltpu.sync_copy(src_hbm.at[pl.ds(base, nh)], srcv)
                pltpu.sync_copy(dst_hbm.at[pl.ds(base, nh)], dstv)

                for p in range(3):
                    pltpu.async_copy(table.at[srcv.at[p]], bufs[p], sg[p])

                def quint(g, __):
                    for p in range(5):
                        j = 5 * g + p
                        q = (p + 3) % 5
                        pltpu.make_async_copy(
                            table.at[srcv.at[j]], bufs[p], sg[p]).wait()
                        pltpu.async_copy(
                            bufs[p], acc.at[dstv.at[j]], ss[p], add=True)

                        @pl.when(j >= 2)
                        def _wait_prev():
                            pltpu.make_async_copy(
                                bufs[q], acc.at[dstv.at[j]], ss[q]).wait()

                        @pl.when(j + 3 < nh)
                        def _issue_next():
                            pltpu.async_copy(
                                table.at[srcv.at[j + 3]], bufs[q], sg[q])
                    return __
                lax.fori_loop(0, nh // 5, quint, None)
                pltpu.make_async_copy(
                    bufs[(nh - 2) % 5], acc.at[dstv.at[nh - 2]], ss[(nh - 2) % 5]).wait()
                pltpu.make_async_copy(
                    bufs[(nh - 1) % 5], acc.at[dstv.at[nh - 1]], ss[(nh - 1) % 5]).wait()
                return _
            lax.fori_loop(0, 2, half_loop, None)
            plsc.subcore_barrier()

            def co(t, _):
                r0 = sid * rows_acc + t * zrows
                pltpu.sync_copy(acc.at[pl.ds(r0, zrows)],
                                out_hbm.at[cid, pl.ds(r0, zrows),
                                           pl.ds(q * 64, 64)])
                return _
            lax.fori_loop(0, nz, co, None)

    return agg_kernel


def _mm1_body(x_ref, w_ref, dg_ref, hs_ref, dinv_ref):
    deg = dg_ref[...] + 1.0
    dinv = lax.rsqrt(jnp.maximum(deg, 1e-12))
    xw = jnp.dot(x_ref[...], w_ref[...], preferred_element_type=F32)
    hs = xw * dinv
    hs_ref[0] = hs[:, :128]
    hs_ref[1] = hs[:, 128:]
    dinv_ref[...] = dinv


def _mm2_body(agg_ref, dinv_ref, b1_ref, w2_ref, hs2_ref):
    dinv = dinv_ref[...]
    hq = [jnp.tanh(dinv * agg_ref[q] + b1_ref[q]) for q in range(2)]
    h = jnp.concatenate(hq, axis=1)
    hw = jnp.dot(h, w2_ref[...], preferred_element_type=F32) * dinv
    hs2_ref[0] = hw[:, :128]
    hs2_ref[1] = hw[:, 128:]


def _fin_body(agg_ref, dinv_ref, b2_ref, fcw_ref, fcb_ref, emb_ref,
              pred_ref):
    dinv = dinv_ref[...]
    eq = [dinv * agg_ref[q] + b2_ref[q] for q in range(2)]
    emb = jnp.concatenate(eq, axis=1)
    emb_ref[...] = emb
    pred_ref[...] = jax.nn.sigmoid(
        jnp.dot(emb, fcw_ref[...], preferred_element_type=F32) + fcb_ref[0, 0])


def kernel(x, edge_index, W1, b1, W2, b2, fcW, fcb):
    n, d = x.shape
    h = W1.shape[1]
    e = edge_index.shape[1]

    n_pad = ((n + 2047) // 2048) * 2048
    ea_q = _NS * _BA * 10
    e_pad_a = ((e + ea_q - 1) // ea_q) * ea_q
    blk = 2000
    grid = (n // blk,)

    src = edge_index[0]
    dst = edge_index[1]
    srcpa = jnp.concatenate([src, jnp.zeros((e_pad_a - e,), I32)])
    dstpa = jnp.concatenate([dst, jnp.full((e_pad_a - e,), n, I32)])
    src2da = srcpa.reshape(e_pad_a // _BA, _BA)
    dst2da = dstpa.reshape(e_pad_a // _BA, _BA)

    deg_call = _make_deg_kernel(n_pad, e_pad_a)
    agg_call = _make_agg_kernel(n, n_pad, e_pad_a)

    degflat = deg_call(dst2da)
    dg = (degflat[:n] + degflat[n_pad:n_pad + n]).reshape(n, 1)

    hs1, dinv = pl.pallas_call(
        _mm1_body,
        grid=grid,
        in_specs=[
            pl.BlockSpec((blk, d), lambda i: (i, 0)),
            pl.BlockSpec((d, h), lambda i: (0, 0)),
            pl.BlockSpec((blk, 1), lambda i: (i, 0)),
        ],
        out_specs=[
            pl.BlockSpec((2, blk, 128), lambda i: (0, i, 0)),
            pl.BlockSpec((blk, 1), lambda i: (i, 0)),
        ],
        out_shape=[
            jax.ShapeDtypeStruct((2, n_pad, 128), F32),
            jax.ShapeDtypeStruct((n, 1), F32),
        ],
    )(x, W1, dg)

    agg1 = agg_call(hs1, src2da, dst2da)

    hs2 = pl.pallas_call(
        _mm2_body,
        grid=grid,
        in_specs=[
            pl.BlockSpec((2, blk, 128), lambda i: (0, i, 0)),
            pl.BlockSpec((blk, 1), lambda i: (i, 0)),
            pl.BlockSpec((2, 1, 128), lambda i: (0, 0, 0)),
            pl.BlockSpec((h, h), lambda i: (0, 0)),
        ],
        out_specs=pl.BlockSpec((2, blk, 128), lambda i: (0, i, 0)),
        out_shape=jax.ShapeDtypeStruct((2, n_pad, 128), F32),
    )(agg1, dinv, b1.reshape(2, 1, 128), W2)

    agg2 = agg_call(hs2, src2da, dst2da)

    emb, pred = pl.pallas_call(
        _fin_body,
        grid=grid,
        in_specs=[
            pl.BlockSpec((2, blk, 128), lambda i: (0, i, 0)),
            pl.BlockSpec((blk, 1), lambda i: (i, 0)),
            pl.BlockSpec((2, 1, 128), lambda i: (0, 0, 0)),
            pl.BlockSpec((h, 1), lambda i: (0, 0)),
            pl.BlockSpec((1, 1), lambda i: (0, 0)),
        ],
        out_specs=[
            pl.BlockSpec((blk, h), lambda i: (i, 0)),
            pl.BlockSpec((blk, 1), lambda i: (i, 0)),
        ],
        out_shape=[
            jax.ShapeDtypeStruct((n, h), F32),
            jax.ShapeDtypeStruct((n, 1), F32),
        ],
    )(agg2, dinv, b2.reshape(2, 1, 128), fcW, fcb.reshape(1, 1))

    return (emb, pred)
